# baseline (speedup 1.0000x reference)
import functools

import jax
import jax.numpy as jnp
from jax import lax
from jax.experimental import pallas as pl
from jax.experimental.pallas import tpu as pltpu
from jax.experimental.pallas import tpu_sc as plsc

N = 10000
E = 320000
D = 128
NC = 2
NS = 16
NW = NC * NS
EPW = E // NW
CF = 128
CK = 320
CHN = 32
EPWP = CK * CHN
EP = NW * EPWP
NP = 10112
RPS = NP // NS

_mesh = plsc.VectorSubcoreMesh(core_axis_name="c", subcore_axis_name="s")


ZF = RPS // CF
ZR = RPS - ZF * CF


@functools.partial(
    pl.kernel,
    out_type=jax.ShapeDtypeStruct((NC, NP, 16), jnp.float32),
    mesh=_mesh,
    scratch_types=[
        pltpu.VMEM((1, CK), jnp.int32),
        pltpu.VMEM((CK, 16), jnp.float32),
        pltpu.VMEM_SHARED((NP, 16), jnp.float32),
    ],
)
def _sc_degree(dst_hbm, z16_hbm, out_hbm, dsti, ones, hist_sh):
    cid = lax.axis_index("c")
    sid = lax.axis_index("s")
    wid = sid * NC + cid

    @pl.loop(0, CK)
    def _(r):
        ones[r, :] = jnp.ones((16,), jnp.float32)

    pltpu.sync_copy(z16_hbm, hist_sh)

    plsc.subcore_barrier()

    @pl.loop(0, CHN)
    def _(j):
        base = wid * EPWP + j * CK
        pltpu.sync_copy(dst_hbm.at[pl.ds(base, CK)], dsti.at[0])
        pltpu.sync_copy(ones, hist_sh.at[dsti.at[0]], add=True)

    plsc.subcore_barrier()
    pltpu.sync_copy(hist_sh, out_hbm.at[cid])


@functools.partial(
    pl.kernel,
    out_type=jax.ShapeDtypeStruct((NC, NP, D), jnp.float32),
    mesh=_mesh,
    scratch_types=[
        pltpu.VMEM((CK,), jnp.int32),
        pltpu.VMEM((1, CK), jnp.int32),
        pltpu.VMEM((ZF, CF), jnp.int32),
        pltpu.VMEM((1, ZR), jnp.int32),
        pltpu.VMEM((CK, D), jnp.float32),
        pltpu.VMEM_SHARED((NP, D), jnp.float32),
        pltpu.SemaphoreType.DMA,
    ],
)
def _sc_scatter(hp_hbm, src_hbm, dst_hbm, rowid_hbm, out_hbm,
                srci, dsti, ridf, ridr, rows, acc_sh, sem):
    cid = lax.axis_index("c")
    sid = lax.axis_index("s")
    wid = sid * NC + cid

    for k in range(ZF):
        pltpu.sync_copy(rowid_hbm.at[pl.ds(sid * RPS + k * CF, CF)],
                        ridf.at[k])
    pltpu.sync_copy(rowid_hbm.at[pl.ds(sid * RPS + ZF * CF, ZR)], ridr.at[0])

    @pl.loop(0, CF)
    def _(r):
        @pl.loop(0, D, step=16)
        def _(c0):
            rows[r, pl.ds(c0, 16)] = jnp.zeros((16,), jnp.float32)

    for k in range(ZF):
        pltpu.sync_copy(rows.at[pl.ds(0, CF)], acc_sh.at[ridf.at[k]])
    pltpu.sync_copy(rows.at[pl.ds(0, ZR)], acc_sh.at[ridr.at[0]])

    plsc.subcore_barrier()

    @pl.loop(0, CHN)
    def _(j):
        base = wid * EPWP + j * CK
        pltpu.sync_copy(src_hbm.at[pl.ds(base, CK)], srci)
        pltpu.sync_copy(dst_hbm.at[pl.ds(base, CK)], dsti.at[0])
        pltpu.async_copy(hp_hbm.at[srci], rows, sem).wait()
        pltpu.sync_copy(rows, acc_sh.at[dsti.at[0]], add=True)

    plsc.subcore_barrier()
    pltpu.sync_copy(acc_sh, out_hbm.at[cid])


def _tc_matmul_body(x_ref, w_ref, o_ref):
    o_ref[...] = jnp.dot(x_ref[...], w_ref[...],
                         preferred_element_type=jnp.float32)


def _tc_matmul(x, w):
    blk = 1000
    return pl.pallas_call(
        _tc_matmul_body,
        grid=(N // blk,),
        in_specs=[pl.BlockSpec((blk, D), lambda i: (i, 0)),
                  pl.BlockSpec((D, D), lambda i: (0, 0))],
        out_specs=pl.BlockSpec((blk, D), lambda i: (i, 0)),
        out_shape=jax.ShapeDtypeStruct((N, D), jnp.float32),
    )(x, w)


def _tc_scale_body(hist_ref, h_ref, o_ref):
    deg = hist_ref[0, :, 0:1] + hist_ref[1, :, 0:1] + 1.0
    dis = lax.rsqrt(deg)
    o_ref[...] = dis * h_ref[...]


def _tc_scale(hist, h):
    blk = 1000
    return pl.pallas_call(
        _tc_scale_body,
        grid=(N // blk,),
        in_specs=[pl.BlockSpec((NC, blk, 16), lambda i: (0, i, 0)),
                  pl.BlockSpec((blk, D), lambda i: (i, 0))],
        out_specs=pl.BlockSpec((blk, D), lambda i: (i, 0)),
        out_shape=jax.ShapeDtypeStruct((N, D), jnp.float32),
    )(hist, h)


def _tc_final_body(acc_ref, hp_ref, hist_ref, b_ref, o_ref):
    deg = hist_ref[0, :, 0:1] + hist_ref[1, :, 0:1] + 1.0
    dis = lax.rsqrt(deg)
    s = acc_ref[0] + acc_ref[1] + hp_ref[...]
    o_ref[...] = jnp.maximum(dis * s + b_ref[...], 0.0)


def _tc_final(acc, hp, hist, b2):
    blk = 1000
    return pl.pallas_call(
        _tc_final_body,
        grid=(N // blk,),
        in_specs=[pl.BlockSpec((NC, blk, D), lambda i: (0, i, 0)),
                  pl.BlockSpec((blk, D), lambda i: (i, 0)),
                  pl.BlockSpec((NC, blk, 16), lambda i: (0, i, 0)),
                  pl.BlockSpec((1, D), lambda i: (0, 0))],
        out_specs=pl.BlockSpec((blk, D), lambda i: (i, 0)),
        out_shape=jax.ShapeDtypeStruct((N, D), jnp.float32),
    )(acc, hp, hist, b2)


def kernel(x, edge_index, W, b, pretrain):
    del pretrain
    src = edge_index[0].astype(jnp.int32)
    dst = edge_index[1].astype(jnp.int32)
    npad = EP - E
    src_f = jnp.concatenate([src, jnp.zeros((npad,), jnp.int32)])
    dst_f = jnp.concatenate(
        [dst, N + (jnp.arange(npad, dtype=jnp.int32) % (NP - N))])
    rowid = jnp.arange(NP, dtype=jnp.int32)
    hist = _sc_degree(dst_f, jnp.zeros((NP, 16), jnp.float32))
    h = _tc_matmul(x, W)
    hp = _tc_scale(hist, h)
    acc = _sc_scatter(hp, src_f, dst_f, rowid)
    return _tc_final(acc, hp, hist, b.reshape(1, D))


# speedup vs baseline: 1.5196x; 1.5196x over previous
import functools

import jax
import jax.numpy as jnp
from jax import lax
from jax.experimental import pallas as pl
from jax.experimental.pallas import tpu as pltpu
from jax.experimental.pallas import tpu_sc as plsc

N = 10000
E = 320000
D = 128
NC = 2
NS = 16
NW = NC * NS
EPW = E // NW
CF = 128
NFULL = EPW // CF
REM = EPW - NFULL * CF
NP = 10112
RPS = NP // NS

_mesh = plsc.VectorSubcoreMesh(core_axis_name="c", subcore_axis_name="s")


ZF = RPS // CF
ZR = RPS - ZF * CF


@functools.partial(
    pl.kernel,
    out_type=jax.ShapeDtypeStruct((NC, NP, 16), jnp.float32),
    mesh=_mesh,
    scratch_types=[
        pltpu.VMEM((1, CF), jnp.int32),
        pltpu.VMEM((1, REM), jnp.int32),
        pltpu.VMEM((CF, 16), jnp.float32),
        pltpu.VMEM_SHARED((NP, 16), jnp.float32),
    ],
)
def _sc_degree(dst_hbm, z16_hbm, out_hbm, dsti, dstir, ones, hist_sh):
    cid = lax.axis_index("c")
    sid = lax.axis_index("s")
    wid = sid * NC + cid

    @pl.loop(0, CF)
    def _(r):
        ones[r, :] = jnp.ones((16,), jnp.float32)

    pltpu.sync_copy(z16_hbm, hist_sh)

    plsc.subcore_barrier()

    @pl.loop(0, NFULL)
    def _(j):
        base = wid * EPW + j * CF
        pltpu.sync_copy(dst_hbm.at[pl.ds(base, CF)], dsti.at[0])
        pltpu.sync_copy(ones, hist_sh.at[dsti.at[0]], add=True)

    base = wid * EPW + NFULL * CF
    pltpu.sync_copy(dst_hbm.at[pl.ds(base, REM)], dstir.at[0])
    pltpu.sync_copy(ones.at[pl.ds(0, REM)], hist_sh.at[dstir.at[0]], add=True)

    plsc.subcore_barrier()
    pltpu.sync_copy(hist_sh, out_hbm.at[cid])


@functools.partial(
    pl.kernel,
    out_type=jax.ShapeDtypeStruct((NC, NP, D), jnp.float32),
    mesh=_mesh,
    scratch_types=[
        pltpu.VMEM((CF,), jnp.int32),
        pltpu.VMEM((REM,), jnp.int32),
        pltpu.VMEM((1, CF), jnp.int32),
        pltpu.VMEM((1, REM), jnp.int32),
        pltpu.VMEM((ZF, CF), jnp.int32),
        pltpu.VMEM((1, ZR), jnp.int32),
        pltpu.VMEM((CF, D), jnp.float32),
        pltpu.VMEM_SHARED((NP, D), jnp.float32),
        pltpu.SemaphoreType.DMA,
    ],
)
def _sc_scatter(hp_hbm, src_hbm, dst_hbm, rowid_hbm, out_hbm,
                srci, srcir, dsti, dstir, ridf, ridr, rows, acc_sh, sem):
    cid = lax.axis_index("c")
    sid = lax.axis_index("s")
    wid = sid * NC + cid

    for k in range(ZF):
        pltpu.sync_copy(rowid_hbm.at[pl.ds(sid * RPS + k * CF, CF)],
                        ridf.at[k])
    pltpu.sync_copy(rowid_hbm.at[pl.ds(sid * RPS + ZF * CF, ZR)], ridr.at[0])

    @pl.loop(0, CF)
    def _(r):
        @pl.loop(0, D, step=16)
        def _(c0):
            rows[r, pl.ds(c0, 16)] = jnp.zeros((16,), jnp.float32)

    for k in range(ZF):
        pltpu.sync_copy(rows, acc_sh.at[ridf.at[k]])
    pltpu.sync_copy(rows.at[pl.ds(0, ZR)], acc_sh.at[ridr.at[0]])

    plsc.subcore_barrier()

    @pl.loop(0, NFULL)
    def _(j):
        base = wid * EPW + j * CF
        pltpu.sync_copy(src_hbm.at[pl.ds(base, CF)], srci)
        pltpu.sync_copy(dst_hbm.at[pl.ds(base, CF)], dsti.at[0])
        pltpu.async_copy(hp_hbm.at[srci], rows, sem).wait()
        pltpu.sync_copy(rows, acc_sh.at[dsti.at[0]], add=True)

    base = wid * EPW + NFULL * CF
    pltpu.sync_copy(src_hbm.at[pl.ds(base, REM)], srcir)
    pltpu.sync_copy(dst_hbm.at[pl.ds(base, REM)], dstir.at[0])
    pltpu.async_copy(hp_hbm.at[srcir], rows.at[pl.ds(0, REM)], sem).wait()
    pltpu.sync_copy(rows.at[pl.ds(0, REM)], acc_sh.at[dstir.at[0]], add=True)

    plsc.subcore_barrier()
    pltpu.sync_copy(acc_sh, out_hbm.at[cid])


def _tc_matmul_body(x_ref, w_ref, o_ref):
    o_ref[...] = jnp.dot(x_ref[...], w_ref[...],
                         preferred_element_type=jnp.float32)


def _tc_matmul(x, w):
    blk = 1000
    return pl.pallas_call(
        _tc_matmul_body,
        grid=(N // blk,),
        in_specs=[pl.BlockSpec((blk, D), lambda i: (i, 0)),
                  pl.BlockSpec((D, D), lambda i: (0, 0))],
        out_specs=pl.BlockSpec((blk, D), lambda i: (i, 0)),
        out_shape=jax.ShapeDtypeStruct((N, D), jnp.float32),
    )(x, w)


def _tc_scale_body(hist_ref, h_ref, o_ref):
    deg = hist_ref[0, :, 0:1] + hist_ref[1, :, 0:1] + 1.0
    dis = lax.rsqrt(deg)
    o_ref[...] = dis * h_ref[...]


def _tc_scale(hist, h):
    blk = 1000
    return pl.pallas_call(
        _tc_scale_body,
        grid=(N // blk,),
        in_specs=[pl.BlockSpec((NC, blk, 16), lambda i: (0, i, 0)),
                  pl.BlockSpec((blk, D), lambda i: (i, 0))],
        out_specs=pl.BlockSpec((blk, D), lambda i: (i, 0)),
        out_shape=jax.ShapeDtypeStruct((N, D), jnp.float32),
    )(hist, h)


def _tc_final_body(acc_ref, hp_ref, hist_ref, b_ref, o_ref):
    deg = hist_ref[0, :, 0:1] + hist_ref[1, :, 0:1] + 1.0
    dis = lax.rsqrt(deg)
    s = acc_ref[0] + acc_ref[1] + hp_ref[...]
    o_ref[...] = jnp.maximum(dis * s + b_ref[...], 0.0)


def _tc_final(acc, hp, hist, b2):
    blk = 1000
    return pl.pallas_call(
        _tc_final_body,
        grid=(N // blk,),
        in_specs=[pl.BlockSpec((NC, blk, D), lambda i: (0, i, 0)),
                  pl.BlockSpec((blk, D), lambda i: (i, 0)),
                  pl.BlockSpec((NC, blk, 16), lambda i: (0, i, 0)),
                  pl.BlockSpec((1, D), lambda i: (0, 0))],
        out_specs=pl.BlockSpec((blk, D), lambda i: (i, 0)),
        out_shape=jax.ShapeDtypeStruct((N, D), jnp.float32),
    )(acc, hp, hist, b2)


def kernel(x, edge_index, W, b, pretrain):
    del pretrain
    src = edge_index[0].astype(jnp.int32)
    dst = edge_index[1].astype(jnp.int32)
    rowid = jnp.arange(NP, dtype=jnp.int32)
    hist = _sc_degree(dst, jnp.zeros((NP, 16), jnp.float32))
    h = _tc_matmul(x, W)
    hp = _tc_scale(hist, h)
    acc = _sc_scatter(hp, src, dst, rowid)
    return _tc_final(acc, hp, hist, b.reshape(1, D))
